# baseline (speedup 1.0000x reference)
import jax
import jax.numpy as jnp
from jax.experimental import pallas as pl
from jax.experimental.pallas import tpu as pltpu

_B, _C, _H, _W = 8, 96, 56, 56
_N, _VD, _K = 1024, 128, 8
_WP = 58
_RQ = _H * _WP
_PAD = 3368


def _nva_block(x2_ref, w_ref, b_ref, v_ref, o_ref, xp_ref, d2_ref):
    s = pl.program_id(0)

    @pl.when(s == 0)
    def _zero_pad_buffer():
        xp_ref[...] = jnp.zeros((_C, _PAD), jnp.float32)

    @pl.when(s < _B)
    def _produce():
        for h in range(_H):
            xp_ref[:, pl.ds((h + 1) * _WP + 1, _W)] = \
                x2_ref[0][:, h * _W:(h + 1) * _W]
        xcat = jnp.concatenate(
            [xp_ref[:, pl.ds(kh * _WP + kw, _RQ)].astype(jnp.bfloat16)
             for kh in range(3) for kw in range(3)],
            axis=0,
        )
        d = jax.lax.dot_general(
            w_ref[...], xcat,
            (((1,), (0,)), ((), ())),
            preferred_element_type=jnp.float32,
            precision=jax.lax.Precision.DEFAULT,
        )
        d2_ref[pl.ds(jax.lax.rem(s, 2), 1)] = (d + b_ref[...])[None]

    @pl.when(s > 0)
    def _consume():
        d = d2_ref[pl.ds(jax.lax.rem(s + 1, 2), 1)][0]
        m = jnp.min(d, axis=0, keepdims=True)
        for _ in range(_K - 1):
            m = jnp.min(jnp.where(d > m, d, jnp.inf), axis=0, keepdims=True)
        mask = (d <= m).astype(jnp.bfloat16)
        o_t = jax.lax.dot_general(
            v_ref[...], mask,
            (((1,), (0,)), ((), ())),
            preferred_element_type=jnp.float32,
            precision=jax.lax.Precision.DEFAULT,
        )
        for h in range(_H):
            o_ref[0, :, pl.ds(h * _W, _W)] = o_t[:, h * _WP:h * _WP + _W]


def kernel(x, centroids, values):
    x2 = x.reshape(_B, _C, _H * _W)
    wt = jnp.transpose(-centroids, (0, 2, 3, 1)).reshape(_N, 9 * _C)
    wt = wt.astype(jnp.bfloat16)
    bias = 0.5 * jnp.sum(centroids.reshape(_N, -1) ** 2, axis=1)[:, None]
    vs = (values.T * (1.0 / _K)).astype(jnp.bfloat16)

    out = pl.pallas_call(
        _nva_block,
        grid=(_B + 1,),
        in_specs=[
            pl.BlockSpec((1, _C, _H * _W), lambda s: (jnp.minimum(s, _B - 1), 0, 0)),
            pl.BlockSpec((_N, 9 * _C), lambda s: (0, 0)),
            pl.BlockSpec((_N, 1), lambda s: (0, 0)),
            pl.BlockSpec((_VD, _N), lambda s: (0, 0)),
        ],
        out_specs=pl.BlockSpec(
            (1, _VD, _H * _W),
            lambda s: (jnp.maximum(s - 1, 0), 0, 0),
        ),
        out_shape=jax.ShapeDtypeStruct((_B, _VD, _H * _W), jnp.float32),
        scratch_shapes=[
            pltpu.VMEM((_C, _PAD), jnp.float32),
            pltpu.VMEM((2, _N, _RQ), jnp.float32),
        ],
        compiler_params=pltpu.CompilerParams(
            dimension_semantics=("arbitrary",),
        ),
    )(x2, wt, bias, vs)

    return out.reshape(_B, _VD, _H, _W)


# speedup vs baseline: 1.0487x; 1.0487x over previous
import jax
import jax.numpy as jnp
from jax.experimental import pallas as pl
from jax.experimental.pallas import tpu as pltpu

_B, _C, _H, _W = 8, 96, 56, 56
_N, _VD, _K = 1024, 128, 8
_WP = 58
_RQ = _H * _WP
_PAD = 3368


def _nva_block(xt_ref, w_ref, b_ref, v_ref, o_ref, xp_ref, d2_ref):
    s = pl.program_id(0)

    @pl.when(s == 0)
    def _zero_pad_buffer():
        xp_ref[...] = jnp.zeros((_PAD, _C), jnp.float32)

    @pl.when(s < _B)
    def _produce():
        xt = jnp.transpose(xt_ref[0])
        for h in range(_H):
            xp_ref[pl.ds((h + 1) * _WP + 1, _W), :] = xt[h * _W:(h + 1) * _W, :]
        xcat = jnp.concatenate(
            [xp_ref[pl.ds(kh * _WP + kw, _RQ), :].astype(jnp.bfloat16)
             for kh in range(3) for kw in range(3)],
            axis=1,
        )
        d = jax.lax.dot_general(
            xcat, w_ref[...],
            (((1,), (0,)), ((), ())),
            preferred_element_type=jnp.float32,
            precision=jax.lax.Precision.DEFAULT,
        )
        d2_ref[pl.ds(jax.lax.rem(s, 2), 1)] = (d + b_ref[...])[None]

    @pl.when(s > 0)
    def _consume():
        d = d2_ref[pl.ds(jax.lax.rem(s + 1, 2), 1)][0]
        m = jnp.min(d, axis=1, keepdims=True)
        for _ in range(_K - 1):
            m = jnp.min(jnp.where(d > m, d, jnp.inf), axis=1, keepdims=True)
        mask = (d <= m).astype(jnp.bfloat16)
        o_t = jax.lax.dot_general(
            v_ref[...], mask,
            (((0,), (1,)), ((), ())),
            preferred_element_type=jnp.float32,
            precision=jax.lax.Precision.DEFAULT,
        )
        for h in range(_H):
            o_ref[0, :, pl.ds(h * _W, _W)] = o_t[:, h * _WP:h * _WP + _W]


def kernel(x, centroids, values):
    xt = x.reshape(_B, _C, _H * _W)
    wt = jnp.transpose(-centroids, (2, 3, 1, 0)).reshape(9 * _C, _N)
    wt = wt.astype(jnp.bfloat16)
    bias = 0.5 * jnp.sum(centroids.reshape(_N, -1) ** 2, axis=1)[None, :]
    vs = (values * (1.0 / _K)).astype(jnp.bfloat16)

    out = pl.pallas_call(
        _nva_block,
        grid=(_B + 1,),
        in_specs=[
            pl.BlockSpec((1, _C, _H * _W), lambda s: (jnp.minimum(s, _B - 1), 0, 0)),
            pl.BlockSpec((9 * _C, _N), lambda s: (0, 0)),
            pl.BlockSpec((1, _N), lambda s: (0, 0)),
            pl.BlockSpec((_N, _VD), lambda s: (0, 0)),
        ],
        out_specs=pl.BlockSpec(
            (1, _VD, _H * _W),
            lambda s: (jnp.maximum(s - 1, 0), 0, 0),
        ),
        out_shape=jax.ShapeDtypeStruct((_B, _VD, _H * _W), jnp.float32),
        scratch_shapes=[
            pltpu.VMEM((_PAD, _C), jnp.float32),
            pltpu.VMEM((2, _RQ, _N), jnp.float32),
        ],
        compiler_params=pltpu.CompilerParams(
            dimension_semantics=("arbitrary",),
        ),
    )(xt, wt, bias, vs)

    return out.reshape(_B, _VD, _H, _W)


# speedup vs baseline: 1.1813x; 1.1264x over previous
import jax
import jax.numpy as jnp
from jax.experimental import pallas as pl
from jax.experimental.pallas import tpu as pltpu

_B, _C, _H, _W = 8, 96, 56, 56
_N, _VD, _K = 1024, 128, 8
_WP = 58
_RQ = _H * _WP
_PAD = 3368


def _nva_block(xt_ref, w_ref, b_ref, v_ref, o_ref, xp_ref, d2_ref):
    s = pl.program_id(0)

    @pl.when(s == 0)
    def _zero_pad_buffer():
        xp_ref[...] = jnp.zeros((_PAD, _C), jnp.float32)

    @pl.when(s < _B)
    def _produce():
        for h in range(_H):
            xp_ref[pl.ds((h + 1) * _WP + 1, _W), :] = xt_ref[0, h]
        xcat = jnp.concatenate(
            [xp_ref[pl.ds(kh * _WP + kw, _RQ), :].astype(jnp.bfloat16)
             for kh in range(3) for kw in range(3)],
            axis=1,
        )
        d = jax.lax.dot_general(
            xcat, w_ref[...],
            (((1,), (0,)), ((), ())),
            preferred_element_type=jnp.float32,
            precision=jax.lax.Precision.DEFAULT,
        )
        d2_ref[pl.ds(jax.lax.rem(s, 2), 1)] = (d + b_ref[...])[None]

    @pl.when(s > 0)
    def _consume():
        d = d2_ref[pl.ds(jax.lax.rem(s + 1, 2), 1)][0]
        m = jnp.min(d, axis=1, keepdims=True)
        for _ in range(_K - 1):
            m = jnp.min(jnp.where(d > m, d, jnp.inf), axis=1, keepdims=True)
        mask = (d <= m).astype(jnp.bfloat16)
        o_t = jax.lax.dot_general(
            v_ref[...], mask,
            (((0,), (1,)), ((), ())),
            preferred_element_type=jnp.float32,
            precision=jax.lax.Precision.DEFAULT,
        )
        for h in range(_H):
            o_ref[0, :, pl.ds(h * _W, _W)] = o_t[:, h * _WP:h * _WP + _W]


def kernel(x, centroids, values):
    xt = jnp.transpose(x, (0, 2, 3, 1))
    wt = jnp.transpose(-centroids, (2, 3, 1, 0)).reshape(9 * _C, _N)
    wt = wt.astype(jnp.bfloat16)
    bias = 0.5 * jnp.sum(centroids.reshape(_N, -1) ** 2, axis=1)[None, :]
    vs = (values * (1.0 / _K)).astype(jnp.bfloat16)

    out = pl.pallas_call(
        _nva_block,
        grid=(_B + 1,),
        in_specs=[
            pl.BlockSpec((1, _H, _W, _C), lambda s: (jnp.minimum(s, _B - 1), 0, 0, 0)),
            pl.BlockSpec((9 * _C, _N), lambda s: (0, 0)),
            pl.BlockSpec((1, _N), lambda s: (0, 0)),
            pl.BlockSpec((_N, _VD), lambda s: (0, 0)),
        ],
        out_specs=pl.BlockSpec(
            (1, _VD, _H * _W),
            lambda s: (jnp.maximum(s - 1, 0), 0, 0),
        ),
        out_shape=jax.ShapeDtypeStruct((_B, _VD, _H * _W), jnp.float32),
        scratch_shapes=[
            pltpu.VMEM((_PAD, _C), jnp.float32),
            pltpu.VMEM((2, _RQ, _N), jnp.float32),
        ],
        compiler_params=pltpu.CompilerParams(
            dimension_semantics=("arbitrary",),
        ),
    )(xt, wt, bias, vs)

    return out.reshape(_B, _VD, _H, _W)
